# Initial kernel scaffold; baseline (speedup 1.0000x reference)
import jax
import jax.numpy as jnp
from jax.experimental import pallas as pl


def kernel(x):
    raise NotImplementedError("write your pallas kernel here")


# speedup vs baseline: 2.0987x; 2.0987x over previous
import functools

import jax
import jax.numpy as jnp
from jax import lax
from jax.experimental import pallas as pl
from jax.experimental.pallas import tpu as pltpu
from jax.experimental.pallas import tpu_sc as plsc

ROWS = 128
COLS = 32768
KTOP = 64
NC = 2
NS = 16
L = 16
NW = NC * NS
RPW = ROWS // NW
NV = COLS // L

NB = 2048
HSTRIDE = NB + 1
HSIZE = L * HSTRIDE

import numpy as np

SAT2 = np.uint32(1 << 21)
SAT3 = np.uint32(1 << 10)
MININT = np.int32(-2147483648)
BIG = np.int32(2147483647)


def _key_of(v):
    s = v >> 31
    return plsc.bitcast(v ^ (s | MININT), jnp.uint32)


def _topk_body(x_hbm, vals_hbm, idx_hbm, vbuf, ibuf, hist, wkey, widx,
               ovals, oidxb):
    lane = lax.iota(jnp.int32, L)
    lanebase = lane * HSTRIDE
    ones = jnp.ones((L,), jnp.int32)
    negones = jnp.full((L,), -1, jnp.int32)
    zeros = jnp.zeros((L,), jnp.int32)
    wid = lax.axis_index("s") * NC + lax.axis_index("c")

    def _zero(i, c):
        hist[pl.ds(i * L, L)] = zeros
        return c
    lax.fori_loop(0, HSIZE // L, _zero, 0)

    def _walk(n_chunks, cook):
        def cond(st):
            j, found = st[0], st[1]
            return jnp.logical_and(j >= 0, jnp.logical_not(found))

        def body(st):
            j, found, b, a, cum = st
            t = cook(j)
            trev = lax.rev(t, (0,))
            c = plsc.cumsum(trev)
            ok = (cum + c) >= KTOP
            hits = jnp.sum(ok.astype(jnp.int32))
            found_here = hits > 0
            i = L - hits
            ci = jnp.sum(jnp.where(lane == i, c, 0))
            ti = jnp.sum(jnp.where(lane == i, trev, 0))
            total = jnp.sum(t)
            b_new = jnp.where(found_here, j * L + (L - 1) - i, b)
            a_new = jnp.where(found_here, cum + ci - ti, a)
            cum_new = jnp.where(found_here, cum, cum + total)
            return (j - 1, jnp.logical_or(found, found_here), b_new, a_new,
                    cum_new)

        st = lax.while_loop(cond, body, (jnp.int32(n_chunks - 1),
                                         jnp.bool_(False), jnp.int32(0),
                                         jnp.int32(0), jnp.int32(0)))
        return st[2], st[3]

    def _hist_chunk(j):
        base = j * L
        t = hist[pl.ds(base, L)]
        for l in range(1, L):
            t = t + hist[pl.ds(l * HSTRIDE + base, L)]
        return t

    def _row(r, carry):
        row = wid * RPW + r
        pltpu.sync_copy(x_hbm.at[pl.ds(row * COLS, COLS)], vbuf)

        def p1(i, c):
            v = vbuf[pl.ds(i * L, L)]
            ku = _key_of(v)
            bkt = plsc.bitcast(ku >> 21, jnp.int32)
            plsc.addupdate_scatter(hist, [lanebase + bkt], ones)
            return c
        lax.fori_loop(0, NV, p1, 0)

        b1, _a1 = _walk(NB // L, _hist_chunk)
        base1u = b1.astype(jnp.uint32) << 21

        def p2(i, off):
            v = vbuf[pl.ds(i * L, L)]
            ku = _key_of(v)
            bkt = plsc.bitcast(ku >> 21, jnp.int32)
            plsc.addupdate_scatter(hist, [lanebase + bkt], negones)
            sel = ku >= base1u
            csum = plsc.cumsum(sel.astype(jnp.int32))
            dst = off + csum - 1
            plsc.store_scatter(vbuf, [dst], plsc.bitcast(ku, jnp.int32),
                               mask=sel)
            plsc.store_scatter(ibuf, [dst], lane + i * L, mask=sel)
            return off + plsc.all_reduce_population_count(sel)
        off = lax.fori_loop(0, NV, p2, zeros)
        n2 = jnp.max(off)
        nv2 = (n2 + (L - 1)) // L

        def l2h(i, c):
            ku = plsc.bitcast(vbuf[pl.ds(i * L, L)], jnp.uint32)
            valid = (i * L + lane) < n2
            key2 = jnp.minimum(ku - base1u, SAT2)
            bkt = plsc.bitcast(jnp.minimum(key2 >> 10, np.uint32(NB - 1)),
                               jnp.int32)
            plsc.addupdate_scatter(hist, [lanebase + bkt], ones, mask=valid)
            return c
        lax.fori_loop(0, nv2, l2h, 0)

        b2, _a2 = _walk(NB // L, _hist_chunk)
        base2u = (b2.astype(jnp.uint32)) << 10

        def l2c(i, off2):
            k = vbuf[pl.ds(i * L, L)]
            ku = plsc.bitcast(k, jnp.uint32)
            valid = (i * L + lane) < n2
            key2 = jnp.minimum(ku - base1u, SAT2)
            bkt = plsc.bitcast(jnp.minimum(key2 >> 10, np.uint32(NB - 1)),
                               jnp.int32)
            plsc.addupdate_scatter(hist, [lanebase + bkt], negones, mask=valid)
            sel = jnp.logical_and(valid, key2 >= base2u)
            csum = plsc.cumsum(sel.astype(jnp.int32))
            dst = off2 + csum - 1
            plsc.store_scatter(vbuf, [dst], k, mask=sel)
            idx = ibuf[pl.ds(i * L, L)]
            plsc.store_scatter(ibuf, [dst], idx, mask=sel)
            return off2 + plsc.all_reduce_population_count(sel)
        off2 = lax.fori_loop(0, nv2, l2c, zeros)
        n3 = jnp.max(off2)
        nv3 = (n3 + (L - 1)) // L

        def l3h(i, c):
            ku = plsc.bitcast(vbuf[pl.ds(i * L, L)], jnp.uint32)
            valid = (i * L + lane) < n3
            key2 = jnp.minimum(ku - base1u, SAT2)
            key3 = jnp.minimum(key2 - base2u, SAT3)
            bkt = plsc.bitcast(key3, jnp.int32)
            plsc.addupdate_scatter(hist, [lanebase + bkt], ones, mask=valid)
            return c
        lax.fori_loop(0, nv3, l3h, 0)

        b3, a3 = _walk((1024 + L) // L + 1, _hist_chunk)
        b3u = b3.astype(jnp.uint32)
        need = KTOP - a3

        def l3c(i, carry):
            off3, eqc = carry
            k = vbuf[pl.ds(i * L, L)]
            ku = plsc.bitcast(k, jnp.uint32)
            valid = (i * L + lane) < n3
            key2 = jnp.minimum(ku - base1u, SAT2)
            key3 = jnp.minimum(key2 - base2u, SAT3)
            bkt = plsc.bitcast(key3, jnp.int32)
            plsc.addupdate_scatter(hist, [lanebase + bkt], negones, mask=valid)
            gt = jnp.logical_and(valid, key3 > b3u)
            eq = jnp.logical_and(valid, key3 == b3u)
            ceq = plsc.cumsum(eq.astype(jnp.int32)) + eqc
            sel = jnp.logical_or(gt, jnp.logical_and(eq, ceq <= need))
            csum = plsc.cumsum(sel.astype(jnp.int32))
            dst = off3 + csum - 1
            plsc.store_scatter(wkey, [dst], k, mask=sel)
            idx = ibuf[pl.ds(i * L, L)]
            plsc.store_scatter(widx, [dst], idx, mask=sel)
            return (off3 + plsc.all_reduce_population_count(sel),
                    eqc + plsc.all_reduce_population_count(eq))
        lax.fori_loop(0, nv3, l3c, (zeros, zeros))

        lane0 = lane == 0
        kk = tuple(plsc.bitcast(wkey[pl.ds(t * L, L)], jnp.uint32)
                   for t in range(KTOP // L))
        ii = tuple(widx[pl.ds(t * L, L)] for t in range(KTOP // L))

        def srt(j, st):
            k0, k1, k2, k3, i0, i1, i2, i3 = st
            m = jnp.maximum(jnp.maximum(k0, k1), jnp.maximum(k2, k3))
            ms = jnp.max(m)
            c0 = jnp.where(k0 == ms, i0, BIG)
            c1 = jnp.where(k1 == ms, i1, BIG)
            c2 = jnp.where(k2 == ms, i2, BIG)
            c3 = jnp.where(k3 == ms, i3, BIG)
            imin = jnp.min(jnp.minimum(jnp.minimum(c0, c1),
                                       jnp.minimum(c2, c3)))
            mv = jnp.full((L,), ms, jnp.uint32)
            ki = plsc.bitcast(mv, jnp.int32)
            u = ki ^ (jnp.invert(ki >> 31) | MININT)
            jv = jnp.full((L,), j, jnp.int32)
            plsc.store_scatter(ovals, [jv], plsc.bitcast(u, jnp.float32),
                               mask=lane0)
            plsc.store_scatter(oidxb, [jv], jnp.full((L,), imin, jnp.int32),
                               mask=lane0)
            h0 = i0 == imin
            h1 = i1 == imin
            h2 = i2 == imin
            h3 = i3 == imin
            return (jnp.where(h0, np.uint32(0), k0),
                    jnp.where(h1, np.uint32(0), k1),
                    jnp.where(h2, np.uint32(0), k2),
                    jnp.where(h3, np.uint32(0), k3),
                    jnp.where(h0, BIG, i0), jnp.where(h1, BIG, i1),
                    jnp.where(h2, BIG, i2), jnp.where(h3, BIG, i3))
        lax.fori_loop(0, KTOP, srt, kk + ii)

        pltpu.sync_copy(ovals, vals_hbm.at[pl.ds(row * KTOP, KTOP)])
        pltpu.sync_copy(oidxb, idx_hbm.at[pl.ds(row * KTOP, KTOP)])
        return carry

    lax.fori_loop(0, RPW, _row, 0)


_topk_sc = functools.partial(
    pl.kernel,
    out_type=(jax.ShapeDtypeStruct((ROWS * KTOP,), jnp.float32),
              jax.ShapeDtypeStruct((ROWS * KTOP,), jnp.int32)),
    mesh=plsc.VectorSubcoreMesh(core_axis_name="c", subcore_axis_name="s",
                                num_cores=NC, num_subcores=NS),
    compiler_params=pltpu.CompilerParams(needs_layout_passes=False),
    scratch_types=[
        pltpu.VMEM((COLS,), jnp.int32),
        pltpu.VMEM((COLS,), jnp.int32),
        pltpu.VMEM((HSIZE,), jnp.int32),
        pltpu.VMEM((KTOP,), jnp.int32),
        pltpu.VMEM((KTOP,), jnp.int32),
        pltpu.VMEM((KTOP,), jnp.float32),
        pltpu.VMEM((KTOP,), jnp.int32),
    ],
)(_topk_body)


def kernel(x):
    xi = lax.bitcast_convert_type(x, jnp.int32).reshape((ROWS * COLS,))
    vals, idx = _topk_sc(xi)
    return vals.reshape((ROWS, KTOP)), idx.reshape((ROWS, KTOP))


# speedup vs baseline: 2.1329x; 1.0163x over previous
import functools

import jax
import jax.numpy as jnp
import numpy as np
from jax import lax
from jax.experimental import pallas as pl
from jax.experimental.pallas import tpu as pltpu
from jax.experimental.pallas import tpu_sc as plsc

ROWS = 128
COLS = 32768
KTOP = 64
NC = 2
NS = 16
L = 16
NW = NC * NS
RPW = ROWS // NW
NV = COLS // L

NB = 2048
HSTRIDE = NB + 1
HSIZE = L * HSTRIDE
HALLOC = 33024

U1 = 8
U2 = 4
UZ = 16
US = 2

SAT2 = np.uint32(1 << 21)
SAT3 = np.uint32(1 << 10)
MININT = np.int32(-2147483648)
BIG = np.int32(2147483647)


def _key_of(v):
    s = v >> 31
    return plsc.bitcast(v ^ (s | MININT), jnp.uint32)


def _topk_body(x_hbm, vals_hbm, idx_hbm, vbuf, ibuf, hist, wkey, widx,
               ovals, oidxb):
    lane = lax.iota(jnp.int32, L)
    lanebase = lane * HSTRIDE
    ones = jnp.ones((L,), jnp.int32)
    negones = jnp.full((L,), -1, jnp.int32)
    zeros = jnp.zeros((L,), jnp.int32)
    wid = lax.axis_index("s") * NC + lax.axis_index("c")

    def _zero(i, c):
        for u in range(UZ):
            hist[pl.ds(i * (L * UZ) + u * L, L)] = zeros
        return c
    lax.fori_loop(0, HALLOC // (L * UZ), _zero, 0)

    def _walk(n_chunks, cook):
        def cond(st):
            j, found = st[0], st[1]
            return jnp.logical_and(j >= 0, jnp.logical_not(found))

        def body(st):
            j, found, b, a, cum = st
            t = cook(j)
            trev = lax.rev(t, (0,))
            c = plsc.cumsum(trev)
            ok = (cum + c) >= KTOP
            hits = jnp.sum(ok.astype(jnp.int32))
            found_here = hits > 0

            def on_found(_):
                i = L - hits
                ci = jnp.sum(jnp.where(lane == i, c, 0))
                ti = jnp.sum(jnp.where(lane == i, trev, 0))
                return (j * L + (L - 1) - i, cum + ci - ti, cum)

            def on_missed(_):
                return (b, a, cum + jnp.sum(t))

            b_new, a_new, cum_new = lax.cond(found_here, on_found, on_missed,
                                             0)
            return (j - 1, jnp.logical_or(found, found_here), b_new, a_new,
                    cum_new)

        st = lax.while_loop(cond, body, (jnp.int32(n_chunks - 1),
                                         jnp.bool_(False), jnp.int32(0),
                                         jnp.int32(0), jnp.int32(0)))
        return st[2], st[3]

    def _hist_chunk(j):
        base = j * L
        t = hist[pl.ds(base, L)]
        for l in range(1, L):
            t = t + hist[pl.ds(l * HSTRIDE + base, L)]
        return t

    def _row(r, carry):
        row = wid * RPW + r
        pltpu.sync_copy(x_hbm.at[pl.ds(row * COLS, COLS)], vbuf)

        def p1(i, c):
            base = i * (L * U1)
            for u in range(U1):
                v = vbuf[pl.ds(base + u * L, L)]
                ku = _key_of(v)
                bkt = plsc.bitcast(ku >> 21, jnp.int32)
                plsc.addupdate_scatter(hist, [lanebase + bkt], ones)
            return c
        lax.fori_loop(0, NV // U1, p1, 0)

        b1, _a1 = _walk(NB // L, _hist_chunk)
        base1u = b1.astype(jnp.uint32) << 21

        def p2(i, off):
            base = i * (L * U2)
            for u in range(U2):
                pos = base + u * L
                v = vbuf[pl.ds(pos, L)]
                ku = _key_of(v)
                bkt = plsc.bitcast(ku >> 21, jnp.int32)
                plsc.addupdate_scatter(hist, [lanebase + bkt], negones)
                sel = ku >= base1u
                csum = plsc.cumsum(sel.astype(jnp.int32))
                dst = off + csum - 1
                plsc.store_scatter(vbuf, [dst], plsc.bitcast(ku, jnp.int32),
                                   mask=sel)
                plsc.store_scatter(ibuf, [dst], lane + pos, mask=sel)
                off = off + plsc.all_reduce_population_count(sel)
            return off
        off = lax.fori_loop(0, NV // U2, p2, zeros)
        n2 = jnp.max(off)
        nv2 = (n2 + (L * US - 1)) // (L * US)

        def l2h(i, c):
            for u in range(US):
                pos = i * (L * US) + u * L
                ku = plsc.bitcast(vbuf[pl.ds(pos, L)], jnp.uint32)
                valid = (pos + lane) < n2
                key2 = jnp.minimum(ku - base1u, SAT2)
                bkt = plsc.bitcast(jnp.minimum(key2 >> 10, np.uint32(NB - 1)),
                                   jnp.int32)
                plsc.addupdate_scatter(hist, [lanebase + bkt], ones,
                                       mask=valid)
            return c
        lax.fori_loop(0, nv2, l2h, 0)

        b2, _a2 = _walk(NB // L, _hist_chunk)
        base2u = (b2.astype(jnp.uint32)) << 10

        def l2c(i, off2):
            for u in range(US):
                pos = i * (L * US) + u * L
                k = vbuf[pl.ds(pos, L)]
                ku = plsc.bitcast(k, jnp.uint32)
                valid = (pos + lane) < n2
                key2 = jnp.minimum(ku - base1u, SAT2)
                bkt = plsc.bitcast(jnp.minimum(key2 >> 10, np.uint32(NB - 1)),
                                   jnp.int32)
                plsc.addupdate_scatter(hist, [lanebase + bkt], negones,
                                       mask=valid)
                sel = jnp.logical_and(valid, key2 >= base2u)
                csum = plsc.cumsum(sel.astype(jnp.int32))
                dst = off2 + csum - 1
                plsc.store_scatter(vbuf, [dst], k, mask=sel)
                idx = ibuf[pl.ds(pos, L)]
                plsc.store_scatter(ibuf, [dst], idx, mask=sel)
                off2 = off2 + plsc.all_reduce_population_count(sel)
            return off2
        off2 = lax.fori_loop(0, nv2, l2c, zeros)
        n3 = jnp.max(off2)
        nv3 = (n3 + (L * US - 1)) // (L * US)

        def l3h(i, c):
            for u in range(US):
                pos = i * (L * US) + u * L
                ku = plsc.bitcast(vbuf[pl.ds(pos, L)], jnp.uint32)
                valid = (pos + lane) < n3
                key2 = jnp.minimum(ku - base1u, SAT2)
                key3 = jnp.minimum(key2 - base2u, SAT3)
                bkt = plsc.bitcast(key3, jnp.int32)
                plsc.addupdate_scatter(hist, [lanebase + bkt], ones,
                                       mask=valid)
            return c
        lax.fori_loop(0, nv3, l3h, 0)

        b3, a3 = _walk((1024 + L) // L + 1, _hist_chunk)
        b3u = b3.astype(jnp.uint32)
        need = KTOP - a3

        def l3c(i, carry3):
            off3, eqc = carry3
            for u in range(US):
                pos = i * (L * US) + u * L
                k = vbuf[pl.ds(pos, L)]
                ku = plsc.bitcast(k, jnp.uint32)
                valid = (pos + lane) < n3
                key2 = jnp.minimum(ku - base1u, SAT2)
                key3 = jnp.minimum(key2 - base2u, SAT3)
                bkt = plsc.bitcast(key3, jnp.int32)
                plsc.addupdate_scatter(hist, [lanebase + bkt], negones,
                                       mask=valid)
                gt = jnp.logical_and(valid, key3 > b3u)
                eq = jnp.logical_and(valid, key3 == b3u)
                ceq = plsc.cumsum(eq.astype(jnp.int32)) + eqc
                sel = jnp.logical_or(gt, jnp.logical_and(eq, ceq <= need))
                csum = plsc.cumsum(sel.astype(jnp.int32))
                dst = off3 + csum - 1
                plsc.store_scatter(wkey, [dst], k, mask=sel)
                idx = ibuf[pl.ds(pos, L)]
                plsc.store_scatter(widx, [dst], idx, mask=sel)
                off3 = off3 + plsc.all_reduce_population_count(sel)
                eqc = eqc + plsc.all_reduce_population_count(eq)
            return (off3, eqc)
        lax.fori_loop(0, nv3, l3c, (zeros, zeros))

        lane0 = lane == 0
        kk = tuple(plsc.bitcast(wkey[pl.ds(t * L, L)], jnp.uint32)
                   for t in range(KTOP // L))
        ii = tuple(widx[pl.ds(t * L, L)] for t in range(KTOP // L))

        def srt(j, st):
            k0, k1, k2, k3, i0, i1, i2, i3 = st
            m = jnp.maximum(jnp.maximum(k0, k1), jnp.maximum(k2, k3))
            ms = jnp.max(m)
            c0 = jnp.where(k0 == ms, i0, BIG)
            c1 = jnp.where(k1 == ms, i1, BIG)
            c2 = jnp.where(k2 == ms, i2, BIG)
            c3 = jnp.where(k3 == ms, i3, BIG)
            imin = jnp.min(jnp.minimum(jnp.minimum(c0, c1),
                                       jnp.minimum(c2, c3)))
            mv = jnp.full((L,), ms, jnp.uint32)
            ki = plsc.bitcast(mv, jnp.int32)
            u = ki ^ (jnp.invert(ki >> 31) | MININT)
            jv = jnp.full((L,), j, jnp.int32)
            plsc.store_scatter(ovals, [jv], plsc.bitcast(u, jnp.float32),
                               mask=lane0)
            plsc.store_scatter(oidxb, [jv], jnp.full((L,), imin, jnp.int32),
                               mask=lane0)
            h0 = i0 == imin
            h1 = i1 == imin
            h2 = i2 == imin
            h3 = i3 == imin
            return (jnp.where(h0, np.uint32(0), k0),
                    jnp.where(h1, np.uint32(0), k1),
                    jnp.where(h2, np.uint32(0), k2),
                    jnp.where(h3, np.uint32(0), k3),
                    jnp.where(h0, BIG, i0), jnp.where(h1, BIG, i1),
                    jnp.where(h2, BIG, i2), jnp.where(h3, BIG, i3))
        lax.fori_loop(0, KTOP, srt, kk + ii)

        pltpu.sync_copy(ovals, vals_hbm.at[pl.ds(row * KTOP, KTOP)])
        pltpu.sync_copy(oidxb, idx_hbm.at[pl.ds(row * KTOP, KTOP)])
        return carry

    lax.fori_loop(0, RPW, _row, 0)


_topk_sc = functools.partial(
    pl.kernel,
    out_type=(jax.ShapeDtypeStruct((ROWS * KTOP,), jnp.float32),
              jax.ShapeDtypeStruct((ROWS * KTOP,), jnp.int32)),
    mesh=plsc.VectorSubcoreMesh(core_axis_name="c", subcore_axis_name="s",
                                num_cores=NC, num_subcores=NS),
    compiler_params=pltpu.CompilerParams(needs_layout_passes=False),
    scratch_types=[
        pltpu.VMEM((COLS,), jnp.int32),
        pltpu.VMEM((COLS,), jnp.int32),
        pltpu.VMEM((HALLOC,), jnp.int32),
        pltpu.VMEM((KTOP,), jnp.int32),
        pltpu.VMEM((KTOP,), jnp.int32),
        pltpu.VMEM((KTOP,), jnp.float32),
        pltpu.VMEM((KTOP,), jnp.int32),
    ],
)(_topk_body)


def kernel(x):
    xi = lax.bitcast_convert_type(x, jnp.int32).reshape((ROWS * COLS,))
    vals, idx = _topk_sc(xi)
    return vals.reshape((ROWS, KTOP)), idx.reshape((ROWS, KTOP))


# speedup vs baseline: 5.7727x; 2.7065x over previous
import functools

import jax
import jax.numpy as jnp
import numpy as np
from jax import lax
from jax.experimental import pallas as pl
from jax.experimental.pallas import tpu as pltpu
from jax.experimental.pallas import tpu_sc as plsc

ROWS = 128
COLS = 32768
KTOP = 64
NC = 2
NS = 16
L = 16
NW = NC * NS
RPW = ROWS // NW
NV = COLS // L

NB = 2048

SAT2 = np.uint32(1 << 21)
SAT3 = np.uint32(1 << 10)
MININT = np.int32(-2147483648)
BIG = np.int32(2147483647)


def _bcast_last(x):
    idx = jnp.full((L, 1), L - 1, jnp.int32)
    dn = lax.GatherDimensionNumbers(offset_dims=(), collapsed_slice_dims=(0,),
                                    start_index_map=(0,))
    return lax.gather(x, idx, dn, (1,),
                      mode=lax.GatherScatterMode.PROMISE_IN_BOUNDS)


def _key_of(vf):
    v = plsc.bitcast(vf, jnp.int32)
    s = v >> 31
    return plsc.bitcast(v ^ (s | MININT), jnp.uint32)


def _slot_of(bkt):
    return ((bkt & 15) << 7) + (bkt >> 4)


def _topk_body(x_hbm, vals_hbm, idx_hbm, vbuf, kcand, icand, hist, wkey,
               widx, ovals, oidxb, dmasem):
    lane = lax.iota(jnp.int32, L)
    lane128 = lane * 128
    ones = jnp.ones((L,), jnp.int32)
    zeros = jnp.zeros((L,), jnp.int32)
    wid = lax.axis_index("s") * NC + lax.axis_index("c")

    def _zero_hist():
        @plsc.parallel_loop(0, NB // L, 1, unroll=4)
        def _z(i):
            hist[pl.ds(i * L, L)] = zeros

    def _walk(n_groups):
        accs = []
        for g in range(n_groups):
            t = hist[pl.ds(g * L, L)]
            for i in range(1, L):
                t = t + hist[pl.ds(i * 128 + g * L, L)]
            accs.append(t)
        S = [None] * n_groups
        carry = zeros
        for g in range(n_groups - 1, -1, -1):
            cs = plsc.cumsum(lax.rev(accs[g], (0,))) + carry
            S[g] = lax.rev(cs, (0,))
            carry = _bcast_last(cs)
        cnt = zeros
        for g in range(n_groups):
            cnt = cnt + plsc.all_reduce_population_count(S[g] >= KTOP)
        J = cnt - 1
        acc_sel = zeros
        for g in range(n_groups):
            chunk_ids = lane + g * L
            acc_sel = acc_sel + jnp.where(chunk_ids == J + 1, S[g], 0)
        cumA = _bcast_last(plsc.cumsum(acc_sel))
        t = plsc.load_gather(hist, [lane128 + J])
        trev = lax.rev(t, (0,))
        c = plsc.cumsum(trev)
        ok = (cumA + c) >= KTOP
        i = 16 - plsc.all_reduce_population_count(ok)
        b = J * L + (L - 1) - i
        ci = _bcast_last(plsc.cumsum(jnp.where(lane == i, c, 0)))
        ti = _bcast_last(plsc.cumsum(jnp.where(lane == i, trev, 0)))
        return b, cumA + ci - ti

    def _row(r, row, cp):
        cp.wait()

        @plsc.parallel_loop(0, NV, 1, unroll=8)
        def _p1(i):
            v = vbuf[pl.ds(i * L, L)]
            ku = _key_of(v)
            bkt = plsc.bitcast(ku >> 21, jnp.int32)
            plsc.addupdate_scatter(hist, [_slot_of(bkt)], ones)

        b1, _a1 = _walk(NB // L // L)
        base1u = plsc.bitcast(b1, jnp.uint32) << 21
        _zero_hist()

        @plsc.parallel_loop(0, NV, 1, unroll=8, carry=zeros)
        def _p2(i, off):
            v = vbuf[pl.ds(i * L, L)]
            ku = _key_of(v)
            sel = ku >= base1u
            csum = plsc.cumsum(sel.astype(jnp.int32))
            dst = off + csum - 1
            plsc.store_scatter(kcand, [dst], plsc.bitcast(ku, jnp.int32),
                               mask=sel)
            plsc.store_scatter(icand, [dst], lane + i * L, mask=sel)
            return off + plsc.all_reduce_population_count(sel)
        n2 = jnp.max(_p2)
        nv2 = (n2 + (L - 1)) // L
        cp_next = (pltpu.async_copy(x_hbm.at[row + 1], vbuf, dmasem)
                   if r + 1 < RPW else None)

        def l2h(i, c):
            ku = plsc.bitcast(kcand[pl.ds(i * L, L)], jnp.uint32)
            valid = (i * L + lane) < n2
            key2 = jnp.minimum(ku - base1u, SAT2)
            bkt = plsc.bitcast(jnp.minimum(key2 >> 10, np.uint32(NB - 1)),
                               jnp.int32)
            plsc.addupdate_scatter(hist, [_slot_of(bkt)], ones, mask=valid)
            return c
        lax.fori_loop(0, nv2, l2h, 0)

        b2, _a2 = _walk(NB // L // L)
        base2u = plsc.bitcast(b2, jnp.uint32) << 10
        _zero_hist()

        def l2c(i, off2):
            k = kcand[pl.ds(i * L, L)]
            ku = plsc.bitcast(k, jnp.uint32)
            valid = (i * L + lane) < n2
            key2 = jnp.minimum(ku - base1u, SAT2)
            sel = jnp.logical_and(valid, key2 >= base2u)
            csum = plsc.cumsum(sel.astype(jnp.int32))
            dst = off2 + csum - 1
            plsc.store_scatter(kcand, [dst], k, mask=sel)
            idx = icand[pl.ds(i * L, L)]
            plsc.store_scatter(icand, [dst], idx, mask=sel)
            return off2 + plsc.all_reduce_population_count(sel)
        off2 = lax.fori_loop(0, nv2, l2c, zeros)
        n3 = jnp.max(off2)
        nv3 = (n3 + (L - 1)) // L

        def l3h(i, c):
            ku = plsc.bitcast(kcand[pl.ds(i * L, L)], jnp.uint32)
            valid = (i * L + lane) < n3
            key2 = jnp.minimum(ku - base1u, SAT2)
            key3 = jnp.minimum(key2 - base2u, SAT3)
            bkt = plsc.bitcast(key3, jnp.int32)
            plsc.addupdate_scatter(hist, [_slot_of(bkt)], ones, mask=valid)
            return c
        lax.fori_loop(0, nv3, l3h, 0)

        b3, a3 = _walk(5)
        b3u = plsc.bitcast(b3, jnp.uint32)
        need = KTOP - a3
        _zero_hist()

        def l3c(i, carry3):
            off3, eqc = carry3
            k = kcand[pl.ds(i * L, L)]
            ku = plsc.bitcast(k, jnp.uint32)
            valid = (i * L + lane) < n3
            key2 = jnp.minimum(ku - base1u, SAT2)
            key3 = jnp.minimum(key2 - base2u, SAT3)
            gt = jnp.logical_and(valid, key3 > b3u)
            eq = jnp.logical_and(valid, key3 == b3u)
            ceq = plsc.cumsum(eq.astype(jnp.int32)) + eqc
            sel = jnp.logical_or(gt, jnp.logical_and(eq, ceq <= need))
            csum = plsc.cumsum(sel.astype(jnp.int32))
            dst = off3 + csum - 1
            plsc.store_scatter(wkey, [dst], k, mask=sel)
            idx = icand[pl.ds(i * L, L)]
            plsc.store_scatter(widx, [dst], idx, mask=sel)
            return (off3 + plsc.all_reduce_population_count(sel),
                    eqc + plsc.all_reduce_population_count(eq))
        lax.fori_loop(0, nv3, l3c, (zeros, zeros))

        lane0 = lane == 0
        kk = tuple(plsc.bitcast(wkey[pl.ds(t * L, L)], jnp.uint32)
                   for t in range(KTOP // L))
        ii = tuple(widx[pl.ds(t * L, L)] for t in range(KTOP // L))

        def srt(j, st):
            k0, k1, k2, k3, i0, i1, i2, i3 = st
            m = jnp.maximum(jnp.maximum(k0, k1), jnp.maximum(k2, k3))
            mv = _bcast_last(plsc.cummax(m))
            c0 = jnp.where(k0 == mv, i0, BIG)
            c1 = jnp.where(k1 == mv, i1, BIG)
            c2 = jnp.where(k2 == mv, i2, BIG)
            c3 = jnp.minimum(jnp.minimum(c0, c1),
                             jnp.minimum(c2, jnp.where(k3 == mv, i3, BIG)))
            iminv = -_bcast_last(plsc.cummax(-c3))
            ki = plsc.bitcast(mv, jnp.int32)
            u = ki ^ (jnp.invert(ki >> 31) | MININT)
            jv = jnp.full((L,), j, jnp.int32)
            plsc.store_scatter(ovals, [jv], plsc.bitcast(u, jnp.float32),
                               mask=lane0)
            plsc.store_scatter(oidxb, [jv], iminv, mask=lane0)
            h0 = i0 == iminv
            h1 = i1 == iminv
            h2 = i2 == iminv
            h3 = i3 == iminv
            return (jnp.where(h0, np.uint32(0), k0),
                    jnp.where(h1, np.uint32(0), k1),
                    jnp.where(h2, np.uint32(0), k2),
                    jnp.where(h3, np.uint32(0), k3),
                    jnp.where(h0, BIG, i0), jnp.where(h1, BIG, i1),
                    jnp.where(h2, BIG, i2), jnp.where(h3, BIG, i3))
        lax.fori_loop(0, KTOP, srt, kk + ii)

        pltpu.sync_copy(ovals, vals_hbm.at[row])
        pltpu.sync_copy(oidxb, idx_hbm.at[row])
        return cp_next

    _zero_hist()
    cp = pltpu.async_copy(x_hbm.at[wid * RPW], vbuf, dmasem)
    for r in range(RPW):
        cp = _row(r, wid * RPW + r, cp)


_topk_sc = functools.partial(
    pl.kernel,
    out_type=(jax.ShapeDtypeStruct((ROWS, KTOP), jnp.float32),
              jax.ShapeDtypeStruct((ROWS, KTOP), jnp.int32)),
    mesh=plsc.VectorSubcoreMesh(core_axis_name="c", subcore_axis_name="s",
                                num_cores=NC, num_subcores=NS),
    compiler_params=pltpu.CompilerParams(needs_layout_passes=False,
                                         use_tc_tiling_on_sc=True),
    scratch_types=[
        pltpu.VMEM((COLS,), jnp.float32),
        pltpu.VMEM((COLS,), jnp.int32),
        pltpu.VMEM((COLS,), jnp.int32),
        pltpu.VMEM((NB,), jnp.int32),
        pltpu.VMEM((KTOP,), jnp.int32),
        pltpu.VMEM((KTOP,), jnp.int32),
        pltpu.VMEM((KTOP,), jnp.float32),
        pltpu.VMEM((KTOP,), jnp.int32),
        pltpu.SemaphoreType.DMA,
    ],
)(_topk_body)


def kernel(x):
    return _topk_sc(x)


# speedup vs baseline: 6.2352x; 1.0801x over previous
import functools

import jax
import jax.numpy as jnp
import numpy as np
from jax import lax
from jax.experimental import pallas as pl
from jax.experimental.pallas import tpu as pltpu
from jax.experimental.pallas import tpu_sc as plsc

ROWS = 128
COLS = 32768
KTOP = 64
NC = 2
NS = 16
L = 16
NW = NC * NS
RPW = ROWS // NW
NV = COLS // L

NB = 2048
HALLOC = NB + L

SAT2 = np.uint32(1 << 21)
SAT3 = np.uint32(1 << 10)
MININT = np.int32(-2147483648)
BIG = np.int32(2147483647)


def _bcast_last(x):
    idx = jnp.full((L, 1), L - 1, jnp.int32)
    dn = lax.GatherDimensionNumbers(offset_dims=(), collapsed_slice_dims=(0,),
                                    start_index_map=(0,))
    return lax.gather(x, idx, dn, (1,),
                      mode=lax.GatherScatterMode.PROMISE_IN_BOUNDS)


def _key_of(vf):
    v = plsc.bitcast(vf, jnp.int32)
    s = v >> 31
    return plsc.bitcast(v ^ (s | MININT), jnp.uint32)


def _topk_body(x_hbm, vals_hbm, idx_hbm, vbuf, kcand, icand, hist, wkey,
               widx, ovals, oidxb, dmasem):
    lane = lax.iota(jnp.int32, L)
    ones = jnp.ones((L,), jnp.int32)
    zeros = jnp.zeros((L,), jnp.int32)
    wid = lax.axis_index("s") * NC + lax.axis_index("c")

    def _zero_hist():
        @plsc.parallel_loop(0, HALLOC // L, 1, unroll=4)
        def _z(i):
            hist[pl.ds(i * L, L)] = zeros

    def _walk(n_chunks):
        def cond(st):
            j, found = st[0], st[1]
            return jnp.logical_and(j >= 0, jnp.logical_not(found))

        def body(st):
            j, found, b, a, cum = st
            t = hist[pl.ds(j * L, L)]
            trev = lax.rev(t, (0,))
            c = plsc.cumsum(trev)
            ok = (cum + c) >= KTOP
            hits = jnp.sum(ok.astype(jnp.int32))
            found_here = hits > 0

            def on_found(_):
                i = L - hits
                ci = jnp.sum(jnp.where(lane == i, c, 0))
                ti = jnp.sum(jnp.where(lane == i, trev, 0))
                return (j * L + (L - 1) - i, cum + ci - ti, cum)

            def on_missed(_):
                return (b, a, cum + jnp.sum(t))

            b_new, a_new, cum_new = lax.cond(found_here, on_found, on_missed,
                                             0)
            return (j - 1, jnp.logical_or(found, found_here), b_new, a_new,
                    cum_new)

        st = lax.while_loop(cond, body, (jnp.int32(n_chunks - 1),
                                         jnp.bool_(False), jnp.int32(0),
                                         jnp.int32(0), jnp.int32(0)))
        return st[2], st[3]

    def _row(r, row, cp, b1_guess):
        cp.wait()

        if b1_guess is None:
            @plsc.parallel_loop(0, NV, 1, unroll=8)
            def _p1(i):
                v = vbuf[pl.ds(i * L, L)]
                ku = _key_of(v)
                bkt = plsc.bitcast(ku >> 21, jnp.int32)
                plsc.addupdate_scatter(hist, [bkt], ones)

            b1, _a1 = _walk(NB // L)
            base1u = b1.astype(jnp.uint32) << 21
            _zero_hist()

            @plsc.parallel_loop(0, NV, 1, unroll=4, carry=zeros)
            def _p2(i, off):
                v = vbuf[pl.ds(i * L, L)]
                ku = _key_of(v)
                sel = ku >= base1u
                csum = plsc.cumsum(sel.astype(jnp.int32))
                dst = off + csum - 1
                plsc.store_scatter(kcand, [dst], plsc.bitcast(ku, jnp.int32),
                                   mask=sel)
                plsc.store_scatter(icand, [dst], lane + i * L, mask=sel)
                return off + plsc.all_reduce_population_count(sel)
            off = _p2
        else:
            t0 = jnp.maximum(b1_guess - 4, 0)
            t0u = t0.astype(jnp.uint32) << 21

            @plsc.parallel_loop(0, NV, 1, unroll=4, carry=zeros)
            def _pf(i, off):
                v = vbuf[pl.ds(i * L, L)]
                ku = _key_of(v)
                bkt = plsc.bitcast(ku >> 21, jnp.int32)
                plsc.addupdate_scatter(hist, [bkt], ones)
                sel = ku >= t0u
                csum = plsc.cumsum(sel.astype(jnp.int32))
                dst = off + csum - 1
                plsc.store_scatter(kcand, [dst], plsc.bitcast(ku, jnp.int32),
                                   mask=sel)
                plsc.store_scatter(icand, [dst], lane + i * L, mask=sel)
                return off + plsc.all_reduce_population_count(sel)

            b1, _a1 = _walk(NB // L)
            base1u = b1.astype(jnp.uint32) << 21
            _zero_hist()
            missed = b1 < t0
            nfb = jnp.where(missed, NV, 0)

            def _pfb(i, offc):
                v = vbuf[pl.ds(i * L, L)]
                ku = _key_of(v)
                sel = ku >= base1u
                csum = plsc.cumsum(sel.astype(jnp.int32))
                dst = offc + csum - 1
                plsc.store_scatter(kcand, [dst], plsc.bitcast(ku, jnp.int32),
                                   mask=sel)
                plsc.store_scatter(icand, [dst], lane + i * L, mask=sel)
                return offc + plsc.all_reduce_population_count(sel)
            off_fb = lax.fori_loop(0, nfb, _pfb, zeros)
            off = jnp.where(missed, off_fb, _pf)

        n2 = jnp.max(off)
        nv2 = (n2 + (L - 1)) // L
        cp_next = (pltpu.async_copy(x_hbm.at[row + 1], vbuf, dmasem)
                   if r + 1 < RPW else None)

        def l2h(i, c):
            ku = plsc.bitcast(kcand[pl.ds(i * L, L)], jnp.uint32)
            valid = jnp.logical_and((i * L + lane) < n2, ku >= base1u)
            key2 = jnp.minimum(ku - base1u, SAT2)
            bkt = plsc.bitcast(jnp.minimum(key2 >> 10, np.uint32(NB - 1)),
                               jnp.int32)
            plsc.addupdate_scatter(hist, [bkt], ones, mask=valid)
            return c
        lax.fori_loop(0, nv2, l2h, 0)

        b2, _a2 = _walk(NB // L)
        base2u = (b2.astype(jnp.uint32)) << 10
        _zero_hist()

        def l2c(i, off2):
            k = kcand[pl.ds(i * L, L)]
            ku = plsc.bitcast(k, jnp.uint32)
            valid = jnp.logical_and((i * L + lane) < n2, ku >= base1u)
            key2 = jnp.minimum(ku - base1u, SAT2)
            sel = jnp.logical_and(valid, key2 >= base2u)
            csum = plsc.cumsum(sel.astype(jnp.int32))
            dst = off2 + csum - 1
            plsc.store_scatter(kcand, [dst], k, mask=sel)
            idx = icand[pl.ds(i * L, L)]
            plsc.store_scatter(icand, [dst], idx, mask=sel)
            return off2 + plsc.all_reduce_population_count(sel)
        off2 = lax.fori_loop(0, nv2, l2c, zeros)
        n3 = jnp.max(off2)
        nv3 = (n3 + (L - 1)) // L

        def l3h(i, c):
            ku = plsc.bitcast(kcand[pl.ds(i * L, L)], jnp.uint32)
            valid = (i * L + lane) < n3
            key2 = jnp.minimum(ku - base1u, SAT2)
            key3 = jnp.minimum(key2 - base2u, SAT3)
            bkt = plsc.bitcast(key3, jnp.int32)
            plsc.addupdate_scatter(hist, [bkt], ones, mask=valid)
            return c
        lax.fori_loop(0, nv3, l3h, 0)

        b3, a3 = _walk((1024 + L) // L + 1)
        b3u = b3.astype(jnp.uint32)
        need = KTOP - a3
        _zero_hist()

        def l3c(i, carry3):
            off3, eqc = carry3
            k = kcand[pl.ds(i * L, L)]
            ku = plsc.bitcast(k, jnp.uint32)
            valid = (i * L + lane) < n3
            key2 = jnp.minimum(ku - base1u, SAT2)
            key3 = jnp.minimum(key2 - base2u, SAT3)
            gt = jnp.logical_and(valid, key3 > b3u)
            eq = jnp.logical_and(valid, key3 == b3u)
            ceq = plsc.cumsum(eq.astype(jnp.int32)) + eqc
            sel = jnp.logical_or(gt, jnp.logical_and(eq, ceq <= need))
            csum = plsc.cumsum(sel.astype(jnp.int32))
            dst = off3 + csum - 1
            plsc.store_scatter(wkey, [dst], k, mask=sel)
            idx = icand[pl.ds(i * L, L)]
            plsc.store_scatter(widx, [dst], idx, mask=sel)
            return (off3 + plsc.all_reduce_population_count(sel),
                    eqc + plsc.all_reduce_population_count(eq))
        lax.fori_loop(0, nv3, l3c, (zeros, zeros))

        lane0 = lane == 0
        kk = tuple(plsc.bitcast(wkey[pl.ds(t * L, L)], jnp.uint32)
                   for t in range(KTOP // L))
        ii = tuple(widx[pl.ds(t * L, L)] for t in range(KTOP // L))

        def srt(j, st):
            k0, k1, k2, k3, i0, i1, i2, i3 = st
            m = jnp.maximum(jnp.maximum(k0, k1), jnp.maximum(k2, k3))
            mv = _bcast_last(plsc.cummax(m))
            c0 = jnp.where(k0 == mv, i0, BIG)
            c1 = jnp.where(k1 == mv, i1, BIG)
            c2 = jnp.where(k2 == mv, i2, BIG)
            c3 = jnp.minimum(jnp.minimum(c0, c1),
                             jnp.minimum(c2, jnp.where(k3 == mv, i3, BIG)))
            iminv = -_bcast_last(plsc.cummax(-c3))
            ki = plsc.bitcast(mv, jnp.int32)
            u = ki ^ (jnp.invert(ki >> 31) | MININT)
            jv = jnp.full((L,), j, jnp.int32)
            plsc.store_scatter(ovals, [jv], plsc.bitcast(u, jnp.float32),
                               mask=lane0)
            plsc.store_scatter(oidxb, [jv], iminv, mask=lane0)
            h0 = i0 == iminv
            h1 = i1 == iminv
            h2 = i2 == iminv
            h3 = i3 == iminv
            return (jnp.where(h0, np.uint32(0), k0),
                    jnp.where(h1, np.uint32(0), k1),
                    jnp.where(h2, np.uint32(0), k2),
                    jnp.where(h3, np.uint32(0), k3),
                    jnp.where(h0, BIG, i0), jnp.where(h1, BIG, i1),
                    jnp.where(h2, BIG, i2), jnp.where(h3, BIG, i3))
        lax.fori_loop(0, KTOP, srt, kk + ii)

        pltpu.sync_copy(ovals, vals_hbm.at[row])
        pltpu.sync_copy(oidxb, idx_hbm.at[row])
        return cp_next, b1

    _zero_hist()
    cp = pltpu.async_copy(x_hbm.at[wid * RPW], vbuf, dmasem)
    b1_guess = None
    for r in range(RPW):
        cp, b1_guess = _row(r, wid * RPW + r, cp, b1_guess)


_topk_sc = functools.partial(
    pl.kernel,
    out_type=(jax.ShapeDtypeStruct((ROWS, KTOP), jnp.float32),
              jax.ShapeDtypeStruct((ROWS, KTOP), jnp.int32)),
    mesh=plsc.VectorSubcoreMesh(core_axis_name="c", subcore_axis_name="s",
                                num_cores=NC, num_subcores=NS),
    compiler_params=pltpu.CompilerParams(needs_layout_passes=False,
                                         use_tc_tiling_on_sc=True),
    scratch_types=[
        pltpu.VMEM((COLS,), jnp.float32),
        pltpu.VMEM((COLS,), jnp.int32),
        pltpu.VMEM((COLS,), jnp.int32),
        pltpu.VMEM((HALLOC,), jnp.int32),
        pltpu.VMEM((KTOP,), jnp.int32),
        pltpu.VMEM((KTOP,), jnp.int32),
        pltpu.VMEM((KTOP,), jnp.float32),
        pltpu.VMEM((KTOP,), jnp.int32),
        pltpu.SemaphoreType.DMA,
    ],
)(_topk_body)


def kernel(x):
    return _topk_sc(x)


# speedup vs baseline: 7.0780x; 1.1352x over previous
import functools

import jax
import jax.numpy as jnp
import numpy as np
from jax import lax
from jax.experimental import pallas as pl
from jax.experimental.pallas import tpu as pltpu
from jax.experimental.pallas import tpu_sc as plsc

ROWS = 128
COLS = 32768
KTOP = 64
NC = 2
NS = 16
L = 16
NW = NC * NS
RPW = ROWS // NW
NV = COLS // L

NB = 2048
HALLOC = NB + L

SAT2 = np.uint32(1 << 21)
SAT3 = np.uint32(1 << 10)
MININT = np.int32(-2147483648)
BIG = np.int32(2147483647)


def _bcast_last(x):
    idx = jnp.full((L, 1), L - 1, jnp.int32)
    dn = lax.GatherDimensionNumbers(offset_dims=(), collapsed_slice_dims=(0,),
                                    start_index_map=(0,))
    return lax.gather(x, idx, dn, (1,),
                      mode=lax.GatherScatterMode.PROMISE_IN_BOUNDS)


def _key_of(vf):
    v = plsc.bitcast(vf, jnp.int32)
    s = v >> 31
    return plsc.bitcast(v ^ (s | MININT), jnp.uint32)


def _topk_body(x_hbm, vals_hbm, idx_hbm, vbuf, kcand, icand, hist, wkey,
               widx, ovals, oidxb, dmasem, outsem):
    lane = lax.iota(jnp.int32, L)
    ones = jnp.ones((L,), jnp.int32)
    zeros = jnp.zeros((L,), jnp.int32)
    wid = lax.axis_index("s") * NC + lax.axis_index("c")

    def _zero_hist():
        @plsc.parallel_loop(0, HALLOC // L, 1, unroll=4)
        def _z(i):
            hist[pl.ds(i * L, L)] = zeros

    def _walk(n_chunks):
        def cond(st):
            j, found = st[0], st[1]
            return jnp.logical_and(j >= 0, jnp.logical_not(found))

        def body(st):
            j, found, b, a, cum = st
            t = hist[pl.ds(j * L, L)]
            trev = lax.rev(t, (0,))
            c = plsc.cumsum(trev)
            ok = (cum + c) >= KTOP
            hits = jnp.sum(ok.astype(jnp.int32))
            found_here = hits > 0

            def on_found(_):
                i = L - hits
                ci = jnp.sum(jnp.where(lane == i, c, 0))
                ti = jnp.sum(jnp.where(lane == i, trev, 0))
                return (j * L + (L - 1) - i, cum + ci - ti, cum)

            def on_missed(_):
                return (b, a, cum + jnp.sum(t))

            b_new, a_new, cum_new = lax.cond(found_here, on_found, on_missed,
                                             0)
            return (j - 1, jnp.logical_or(found, found_here), b_new, a_new,
                    cum_new)

        st = lax.while_loop(cond, body, (jnp.int32(n_chunks - 1),
                                         jnp.bool_(False), jnp.int32(0),
                                         jnp.int32(0), jnp.int32(0)))
        return st[2], st[3]

    def _row(r, row, cp, prev_out):

        cp.wait()

        @plsc.parallel_loop(0, NV, 1, unroll=8)
        def _p1(i):
            v = vbuf[pl.ds(i * L, L)]
            ku = _key_of(v)
            bkt = plsc.bitcast(ku >> 21, jnp.int32)
            plsc.addupdate_scatter(hist, [bkt], ones)

        b1, _a1 = _walk(NB // L)
        base1u = b1.astype(jnp.uint32) << 21
        _zero_hist()

        @plsc.parallel_loop(0, NV, 1, unroll=8, carry=zeros)
        def _p2(i, off):
            v = vbuf[pl.ds(i * L, L)]
            ku = _key_of(v)
            sel = ku >= base1u
            csum = plsc.cumsum(sel.astype(jnp.int32))
            dst = off + csum - 1
            plsc.store_scatter(kcand, [dst], plsc.bitcast(ku, jnp.int32),
                               mask=sel)
            plsc.store_scatter(icand, [dst], lane + i * L, mask=sel)
            return off + plsc.all_reduce_population_count(sel)
        n2 = jnp.max(_p2)
        nv2 = (n2 + (L - 1)) // L
        cp_next = (pltpu.async_copy(x_hbm.at[row + 1], vbuf, dmasem)
                   if r + 1 < RPW else None)

        def l2h(i, c):
            ku = plsc.bitcast(kcand[pl.ds(i * L, L)], jnp.uint32)
            valid = (i * L + lane) < n2
            key2 = jnp.minimum(ku - base1u, SAT2)
            bkt = plsc.bitcast(jnp.minimum(key2 >> 10, np.uint32(NB - 1)),
                               jnp.int32)
            plsc.addupdate_scatter(hist, [bkt], ones, mask=valid)
            return c
        lax.fori_loop(0, nv2, l2h, 0)

        b2, _a2 = _walk(NB // L)
        base2u = (b2.astype(jnp.uint32)) << 10
        _zero_hist()

        def l2c(i, off2):
            k = kcand[pl.ds(i * L, L)]
            ku = plsc.bitcast(k, jnp.uint32)
            valid = (i * L + lane) < n2
            key2 = jnp.minimum(ku - base1u, SAT2)
            sel = jnp.logical_and(valid, key2 >= base2u)
            csum = plsc.cumsum(sel.astype(jnp.int32))
            dst = off2 + csum - 1
            plsc.store_scatter(kcand, [dst], k, mask=sel)
            idx = icand[pl.ds(i * L, L)]
            plsc.store_scatter(icand, [dst], idx, mask=sel)
            return off2 + plsc.all_reduce_population_count(sel)
        off2 = lax.fori_loop(0, nv2, l2c, zeros)
        n3 = jnp.max(off2)
        nv3 = (n3 + (L - 1)) // L

        def l3h(i, c):
            ku = plsc.bitcast(kcand[pl.ds(i * L, L)], jnp.uint32)
            valid = (i * L + lane) < n3
            key2 = jnp.minimum(ku - base1u, SAT2)
            key3 = jnp.minimum(key2 - base2u, SAT3)
            bkt = plsc.bitcast(key3, jnp.int32)
            plsc.addupdate_scatter(hist, [bkt], ones, mask=valid)
            return c
        lax.fori_loop(0, nv3, l3h, 0)

        b3, a3 = _walk((1024 + L) // L + 1)
        b3u = b3.astype(jnp.uint32)
        need = KTOP - a3
        _zero_hist()

        def l3c(i, carry3):
            off3, eqc = carry3
            k = kcand[pl.ds(i * L, L)]
            ku = plsc.bitcast(k, jnp.uint32)
            valid = (i * L + lane) < n3
            key2 = jnp.minimum(ku - base1u, SAT2)
            key3 = jnp.minimum(key2 - base2u, SAT3)
            gt = jnp.logical_and(valid, key3 > b3u)
            eq = jnp.logical_and(valid, key3 == b3u)
            ceq = plsc.cumsum(eq.astype(jnp.int32)) + eqc
            sel = jnp.logical_or(gt, jnp.logical_and(eq, ceq <= need))
            csum = plsc.cumsum(sel.astype(jnp.int32))
            dst = off3 + csum - 1
            plsc.store_scatter(wkey, [dst], k, mask=sel)
            idx = icand[pl.ds(i * L, L)]
            plsc.store_scatter(widx, [dst], idx, mask=sel)
            return (off3 + plsc.all_reduce_population_count(sel),
                    eqc + plsc.all_reduce_population_count(eq))
        lax.fori_loop(0, nv3, l3c, (zeros, zeros))

        if prev_out is not None:
            prev_out[0].wait()
            prev_out[1].wait()
        lane0 = lane == 0
        kk = tuple(plsc.bitcast(wkey[pl.ds(t * L, L)], jnp.uint32)
                   for t in range(KTOP // L))
        ii = tuple(widx[pl.ds(t * L, L)] for t in range(KTOP // L))

        def srt(j, st):
            k0, k1, k2, k3, i0, i1, i2, i3 = st
            m = jnp.maximum(jnp.maximum(k0, k1), jnp.maximum(k2, k3))
            mv = _bcast_last(plsc.cummax(m))
            c0 = jnp.where(k0 == mv, i0, BIG)
            c1 = jnp.where(k1 == mv, i1, BIG)
            c2 = jnp.where(k2 == mv, i2, BIG)
            c3 = jnp.minimum(jnp.minimum(c0, c1),
                             jnp.minimum(c2, jnp.where(k3 == mv, i3, BIG)))
            iminv = -_bcast_last(plsc.cummax(-c3))
            ki = plsc.bitcast(mv, jnp.int32)
            u = ki ^ (jnp.invert(ki >> 31) | MININT)
            jv = jnp.full((L,), j, jnp.int32)
            plsc.store_scatter(ovals, [jv], plsc.bitcast(u, jnp.float32),
                               mask=lane0)
            plsc.store_scatter(oidxb, [jv], iminv, mask=lane0)
            h0 = i0 == iminv
            h1 = i1 == iminv
            h2 = i2 == iminv
            h3 = i3 == iminv
            return (jnp.where(h0, np.uint32(0), k0),
                    jnp.where(h1, np.uint32(0), k1),
                    jnp.where(h2, np.uint32(0), k2),
                    jnp.where(h3, np.uint32(0), k3),
                    jnp.where(h0, BIG, i0), jnp.where(h1, BIG, i1),
                    jnp.where(h2, BIG, i2), jnp.where(h3, BIG, i3))
        lax.fori_loop(0, KTOP, srt, kk + ii)

        out_cp = (pltpu.async_copy(ovals, vals_hbm.at[row], outsem),
                  pltpu.async_copy(oidxb, idx_hbm.at[row], outsem))
        return cp_next, out_cp

    _zero_hist()
    cp = pltpu.async_copy(x_hbm.at[wid * RPW], vbuf, dmasem)
    prev = None
    for r in range(RPW):
        cp, prev = _row(r, wid * RPW + r, cp, prev)
    prev[0].wait()
    prev[1].wait()


_topk_sc = functools.partial(
    pl.kernel,
    out_type=(jax.ShapeDtypeStruct((ROWS, KTOP), jnp.float32),
              jax.ShapeDtypeStruct((ROWS, KTOP), jnp.int32)),
    mesh=plsc.VectorSubcoreMesh(core_axis_name="c", subcore_axis_name="s",
                                num_cores=NC, num_subcores=NS),
    compiler_params=pltpu.CompilerParams(needs_layout_passes=False,
                                         use_tc_tiling_on_sc=True),
    scratch_types=[
        pltpu.VMEM((COLS,), jnp.float32),
        pltpu.VMEM((COLS,), jnp.int32),
        pltpu.VMEM((COLS,), jnp.int32),
        pltpu.VMEM((HALLOC,), jnp.int32),
        pltpu.VMEM((KTOP,), jnp.int32),
        pltpu.VMEM((KTOP,), jnp.int32),
        pltpu.VMEM((KTOP,), jnp.float32),
        pltpu.VMEM((KTOP,), jnp.int32),
        pltpu.SemaphoreType.DMA,
        pltpu.SemaphoreType.DMA,
    ],
)(_topk_body)


def kernel(x):
    return _topk_sc(x)
